# Initial kernel scaffold; baseline (speedup 1.0000x reference)
import functools

import jax
import jax.numpy as jnp
from jax import lax
from jax.experimental import pallas as pl
from jax.experimental.pallas import tpu as pltpu
from jax.experimental.pallas import tpu_sc as plsc

N = 10000
NP = 10240
E = 320000
NW = 32
ROWS = NP // NW
CH = 2048
EP = ((E + CH - 1) // CH) * CH
GB = 64
GBG = GB // 16
NGB = CH // GB
D = 128


def _proj(x, W, al, ar):
    H = al.shape[0]
    Din = x.shape[1]
    BN = 256

    def body(x_ref, w_ref, al_ref, ar_ref, h_ref, el_ref, er_ref):
        xb = x_ref[...]
        for hh in range(H):
            hb = jnp.dot(xb, w_ref[:, hh * D:(hh + 1) * D],
                         preferred_element_type=jnp.float32)
            h_ref[hh] = hb
            el_ref[hh] = jnp.sum(hb * al_ref[hh][None], axis=1)
            er_ref[hh] = jnp.sum(hb * ar_ref[hh][None], axis=1)

    return pl.pallas_call(
        body,
        grid=(NP // BN,),
        in_specs=[
            pl.BlockSpec((BN, Din), lambda i: (i, 0)),
            pl.BlockSpec((Din, H * D), lambda i: (0, 0)),
            pl.BlockSpec((H, D), lambda i: (0, 0)),
            pl.BlockSpec((H, D), lambda i: (0, 0)),
        ],
        out_specs=(
            pl.BlockSpec((H, BN, D), lambda i: (0, i, 0)),
            pl.BlockSpec((H, BN), lambda i: (0, i)),
            pl.BlockSpec((H, BN), lambda i: (0, i)),
        ),
        out_shape=(
            jax.ShapeDtypeStruct((H, NP, D), jnp.float32),
            jax.ShapeDtypeStruct((H, NP), jnp.float32),
            jax.ShapeDtypeStruct((H, NP), jnp.float32),
        ),
    )(x, W, al, ar)


def _edge_sc(H, apply_elu, mean_heads):
    Dout = D if mean_heads else H * D
    mesh = plsc.VectorSubcoreMesh(core_axis_name="c", subcore_axis_name="s")
    iota16 = lambda: lax.iota(jnp.int32, 16)

    @functools.partial(
        pl.kernel,
        out_type=jax.ShapeDtypeStruct((NP, Dout), jnp.float32),
        mesh=mesh,
        scratch_types=[
            pltpu.VMEM((NP,), jnp.float32),
            pltpu.VMEM((ROWS,), jnp.float32),
            pltpu.VMEM((ROWS,), jnp.float32),
            pltpu.VMEM((ROWS, D), jnp.float32),
            pltpu.VMEM((CH,), jnp.int32),
            pltpu.VMEM((CH,), jnp.int32),
            pltpu.VMEM((2, GB), jnp.int32),
            pltpu.VMEM((2, GB, D), jnp.float32),
            pltpu.VMEM((GB,), jnp.float32),
            pltpu.VMEM((64,), jnp.int32),
            pltpu.VMEM((H, D), jnp.float32),
            pltpu.SemaphoreType.DMA,
            pltpu.SemaphoreType.DMA,
        ],
    )
    def k(src_hbm, dst_hbm, bounds_hbm, el_hbm, er_hbm, h_hbm, b_hbm, out_hbm,
          elv, erv, denv, acc, srcv, dstv, idxb, gbuf, albuf, boundsv, biasv,
          sem0, sem1):
        sems = (sem0, sem1)
        wid = lax.axis_index("s") * 2 + lax.axis_index("c")
        base = wid * ROWS
        pltpu.sync_copy(bounds_hbm, boundsv)
        pltpu.sync_copy(b_hbm, biasv)
        estart = boundsv[wid]
        eend = boundsv[wid + 1]
        cstart = estart // CH
        cend = (eend + CH - 1) // CH

        def zero_acc():
            z = jnp.zeros((16,), jnp.float32)

            def zb(r, _):
                for c in range(D // 16):
                    acc[r, pl.ds(c * 16, 16)] = z
                return 0

            lax.fori_loop(0, ROWS, zb, 0)

        def stage_chunk(ci):
            cbase = ci * CH
            pltpu.sync_copy(src_hbm.at[pl.ds(cbase, CH)], srcv)
            pltpu.sync_copy(dst_hbm.at[pl.ds(cbase, CH)], dstv)
            return cbase

        def edge_group(cbase, off):
            s16 = srcv[pl.ds(off, 16)]
            d16 = dstv[pl.ds(off, 16)]
            gi = cbase + off + iota16()
            msk = (gi >= estart) & (gi < eend)
            dl = jnp.clip(d16 - base, 0, ROWS - 1)
            sc = jnp.clip(s16, 0, N - 1)
            ev = plsc.load_gather(elv, [sc])
            rv = plsc.load_gather(erv, [dl])
            e = ev + rv
            e = jnp.where(e > 0, e, 0.2 * e)
            return msk, dl, jnp.exp(e)

        def pass_a(_hh):
            z = jnp.zeros((16,), jnp.float32)
            for r in range(ROWS // 16):
                denv[pl.ds(r * 16, 16)] = z

            def chunk_body(ci, _):
                cbase = stage_chunk(ci)
                glo = jnp.maximum(0, (estart - cbase) // 16)
                ghi = jnp.minimum(CH // 16, (eend - cbase + 15) // 16)

                def gb(g, _):
                    msk, dl, ex = edge_group(cbase, g * 16)
                    plsc.addupdate_scatter(denv, [dl], ex, msk)
                    return 0

                lax.fori_loop(glo, ghi, gb, 0)
                return 0

            lax.fori_loop(cstart, cend, chunk_body, 0)

        def issue(hh, gg, slot):
            goff = gg * GB
            for u in range(GBG):
                s16 = srcv[pl.ds(goff + u * 16, 16)]
                idxb[slot, pl.ds(u * 16, 16)] = jnp.clip(s16, 0, N - 1) + hh * NP
            pltpu.async_copy(h_hbm.at[idxb.at[slot]], gbuf.at[slot], sems[slot])

        def wait(slot):
            pltpu.make_async_copy(h_hbm.at[idxb.at[slot]], gbuf.at[slot],
                                  sems[slot]).wait()

        def process(cbase, gg, slot):
            goff = gg * GB
            for u in range(GBG):
                off = goff + u * 16
                msk, dl, ex = edge_group(cbase, off)
                den = plsc.load_gather(denv, [dl])
                albuf[pl.ds(u * 16, 16)] = jnp.where(msk, ex / den, 0.0)
                for j in range(16):
                    a = albuf[u * 16 + j]
                    dls = jnp.clip(dstv[off + j] - base, 0, ROWS - 1)
                    for c in range(D // 16):
                        plsc.addupdate(
                            acc.at[dls, pl.ds(c * 16, 16)],
                            a * gbuf[slot, u * 16 + j, pl.ds(c * 16, 16)])

        def pass_b(hh):
            def chunk_body(ci, _):
                cbase = stage_chunk(ci)
                gglo = jnp.maximum(0, (estart - cbase) // GB)
                gghi = jnp.minimum(NGB, (eend - cbase + GB - 1) // GB)
                for b in range(2):
                    @pl.when(gglo + b < gghi)
                    def _():
                        issue(hh, gglo + b, b)

                def rb(r, _):
                    it = r * 2
                    for b in range(2):
                        gg = gglo + it + b

                        @pl.when(gg < gghi)
                        def _():
                            wait(b)
                            process(cbase, gg, b)

                            @pl.when(gg + 2 < gghi)
                            def _():
                                issue(hh, gg + 2, b)
                    return 0

                nrounds = (gghi - gglo + 1) // 2
                lax.fori_loop(0, nrounds, rb, 0)
                return 0

            lax.fori_loop(cstart, cend, chunk_body, 0)

        def writeback(hh):
            def wb(r, _):
                for c in range(D // 16):
                    v = acc[r, pl.ds(c * 16, 16)] + biasv[hh, pl.ds(c * 16, 16)]
                    if apply_elu:
                        v = jnp.where(v > 0, v, jnp.exp(jnp.minimum(v, 0.0)) - 1.0)
                    acc[r, pl.ds(c * 16, 16)] = v
                return 0

            lax.fori_loop(0, ROWS, wb, 0)
            pltpu.sync_copy(acc, out_hbm.at[pl.ds(base, ROWS),
                                            pl.ds(hh * D, D)])

        def finalize_mean():
            scale = jnp.float32(1.0 / H)

            def wb(r, _):
                for c in range(D // 16):
                    bsum = biasv[0, pl.ds(c * 16, 16)]
                    for hh in range(1, H):
                        bsum = bsum + biasv[hh, pl.ds(c * 16, 16)]
                    acc[r, pl.ds(c * 16, 16)] = (
                        acc[r, pl.ds(c * 16, 16)] + bsum) * scale
                return 0

            lax.fori_loop(0, ROWS, wb, 0)
            pltpu.sync_copy(acc, out_hbm.at[pl.ds(base, ROWS)])

        if mean_heads:
            zero_acc()
        for hh in range(H):
            pltpu.sync_copy(el_hbm.at[hh], elv)
            pltpu.sync_copy(er_hbm.at[hh, pl.ds(base, ROWS)], erv)
            if not mean_heads:
                zero_acc()
            pass_a(hh)
            pass_b(hh)
            if not mean_heads:
                writeback(hh)
        if mean_heads:
            finalize_mean()

    return k


def _layer(x, W, al, ar, b, src_s, dst_s, bounds, apply_elu, mean_heads):
    H = al.shape[0]
    h, el, er = _proj(x, W, al, ar)
    hflat = h.reshape(H * NP, D)
    k = _edge_sc(H, apply_elu, mean_heads)
    return k(src_s, dst_s, bounds, el, er, hflat, b)


def kernel(inputs, edge_index, W1, al1, ar1, b1, W2, al2, ar2, b2,
           W3, al3, ar3, b3):
    src = edge_index[0].astype(jnp.int32)
    dst = edge_index[1].astype(jnp.int32)
    order = jnp.argsort(dst)
    src_s = jnp.concatenate(
        [src[order], jnp.zeros((EP - E,), jnp.int32)])
    dst_s0 = dst[order]
    dst_s = jnp.concatenate(
        [dst_s0, jnp.full((EP - E,), N - 1, jnp.int32)])
    bounds = jnp.searchsorted(
        dst_s0, jnp.arange(NW + 1, dtype=jnp.int32) * ROWS).astype(jnp.int32)
    bounds = jnp.concatenate(
        [bounds, jnp.full((64 - NW - 1,), E, jnp.int32)])

    x = jnp.pad(inputs, ((0, NP - N), (0, 0)))
    h1 = _layer(x, W1, al1, ar1, b1, src_s, dst_s, bounds, True, False)
    h2 = _layer(h1, W2, al2, ar2, b2, src_s, dst_s, bounds, True, False)
    h3 = _layer(h2, W3, al3, ar3, b3, src_s, dst_s, bounds, False, True)
    return h3[:N]


# speedup vs baseline: 8.5458x; 8.5458x over previous
import functools

import jax
import jax.numpy as jnp
from jax import lax
from jax.experimental import pallas as pl
from jax.experimental.pallas import tpu as pltpu
from jax.experimental.pallas import tpu_sc as plsc

N = 10000
NP = 10240
E = 320000
NW = 32
ROWS = NP // NW
CH = 2048
EP = ((E + CH - 1) // CH) * CH
GB = 64
GBG = GB // 16
NGB = CH // GB
D = 128


def _proj(x, W, al, ar):
    H = al.shape[0]
    Din = x.shape[1]
    BN = 256

    def body(x_ref, w_ref, al_ref, ar_ref, h_ref, el_ref, er_ref):
        xb = x_ref[...]
        for hh in range(H):
            hb = jnp.dot(xb, w_ref[:, hh * D:(hh + 1) * D],
                         preferred_element_type=jnp.float32)
            h_ref[hh] = hb
            el_ref[hh] = jnp.sum(hb * al_ref[hh][None], axis=1)
            er_ref[hh] = jnp.sum(hb * ar_ref[hh][None], axis=1)

    return pl.pallas_call(
        body,
        grid=(NP // BN,),
        in_specs=[
            pl.BlockSpec((BN, Din), lambda i: (i, 0)),
            pl.BlockSpec((Din, H * D), lambda i: (0, 0)),
            pl.BlockSpec((H, D), lambda i: (0, 0)),
            pl.BlockSpec((H, D), lambda i: (0, 0)),
        ],
        out_specs=(
            pl.BlockSpec((H, BN, D), lambda i: (0, i, 0)),
            pl.BlockSpec((H, BN), lambda i: (0, i)),
            pl.BlockSpec((H, BN), lambda i: (0, i)),
        ),
        out_shape=(
            jax.ShapeDtypeStruct((H, NP, D), jnp.float32),
            jax.ShapeDtypeStruct((H, NP), jnp.float32),
            jax.ShapeDtypeStruct((H, NP), jnp.float32),
        ),
    )(x, W, al, ar)


def _edge_sc(H, apply_elu, mean_heads):
    oshape = (NP, D) if mean_heads else (H, NP, D)
    mesh = plsc.VectorSubcoreMesh(core_axis_name="c", subcore_axis_name="s")
    iota16 = lambda: lax.iota(jnp.int32, 16)

    @functools.partial(
        pl.kernel,
        out_type=jax.ShapeDtypeStruct(oshape, jnp.float32),
        mesh=mesh,
        compiler_params=pltpu.CompilerParams(needs_layout_passes=False),
        scratch_types=[
            pltpu.VMEM((NP,), jnp.float32),
            pltpu.VMEM((ROWS,), jnp.float32),
            pltpu.VMEM((ROWS,), jnp.float32),
            pltpu.VMEM((ROWS, D), jnp.float32),
            pltpu.VMEM((CH,), jnp.int32),
            pltpu.VMEM((CH,), jnp.int32),
            pltpu.VMEM((GB,), jnp.int32),
            pltpu.VMEM((GB, D), jnp.float32),
            pltpu.VMEM((64,), jnp.int32),
            pltpu.VMEM((H * D,), jnp.float32),
            pltpu.SemaphoreType.DMA,
        ],
    )
    def k(src_hbm, dst_hbm, bounds_hbm, el_hbm, er_hbm, h_hbm, b_hbm, out_hbm,
          elv, erv, denv, acc, srcv, dstv, idxv, gbuf, boundsv, biasv,
          sem0):
        wid = lax.axis_index("s") * 2 + lax.axis_index("c")
        base = wid * ROWS
        pltpu.sync_copy(bounds_hbm, boundsv)
        pltpu.sync_copy(b_hbm, biasv)
        bv = boundsv[pl.ds(wid, 16)]
        estart = bv[0]
        eend = bv[1]
        cstart = estart // CH
        cend = (eend + CH - 1) // CH

        def zero_acc():
            z = jnp.zeros((16,), jnp.float32)

            def zb(r, _):
                for c in range(D // 16):
                    acc[r, pl.ds(c * 16, 16)] = z
                return 0

            lax.fori_loop(0, ROWS, zb, 0)

        def stage_chunk(ci):
            cbase = ci * CH
            pltpu.sync_copy(src_hbm.at[pl.ds(cbase, CH)], srcv)
            pltpu.sync_copy(dst_hbm.at[pl.ds(cbase, CH)], dstv)
            return cbase

        def edge_group(cbase, off):
            s16 = srcv[pl.ds(off, 16)]
            d16 = dstv[pl.ds(off, 16)]
            gi = cbase + off + iota16()
            msk = (gi >= estart) & (gi < eend)
            dl = jnp.clip(d16 - base, 0, ROWS - 1)
            sc = jnp.clip(s16, 0, N - 1)
            ev = plsc.load_gather(elv, [sc])
            rv = plsc.load_gather(erv, [dl])
            e = ev + rv
            e = jnp.where(e > 0, e, 0.2 * e)
            return msk, dl, jnp.exp(e)

        def pass_a(_hh):
            z = jnp.zeros((16,), jnp.float32)
            for r in range(ROWS // 16):
                denv[pl.ds(r * 16, 16)] = z

            def chunk_body(ci, _):
                cbase = stage_chunk(ci)
                glo = jnp.maximum(0, (estart - cbase) // 16)
                ghi = jnp.minimum(CH // 16, (eend - cbase + 15) // 16)

                def gb(g, _):
                    msk, dl, ex = edge_group(cbase, g * 16)
                    plsc.addupdate_scatter(denv, [dl], ex, mask=msk)
                    return 0

                lax.fori_loop(glo, ghi, gb, 0)
                return 0

            lax.fori_loop(cstart, cend, chunk_body, 0)

        def process(cbase, gg, hbase):
            goff = gg * GB
            for u in range(GBG):
                s16 = srcv[pl.ds(goff + u * 16, 16)]
                idxv[pl.ds(u * 16, 16)] = jnp.clip(s16, 0, N - 1) + hbase
            pltpu.async_copy(h_hbm.at[idxv], gbuf, sem0).wait()
            for u in range(GBG):
                off = goff + u * 16
                msk, dl, ex = edge_group(cbase, off)
                den = plsc.load_gather(denv, [dl])
                alv = jnp.where(msk, ex / den, 0.0)
                for j in range(16):
                    a = alv[j]
                    dls = dl[j]
                    for c in range(D // 16):
                        plsc.addupdate(
                            acc.at[dls, pl.ds(c * 16, 16)],
                            a * gbuf[u * 16 + j, pl.ds(c * 16, 16)])

        def pass_b(hh):
            hbase = hh * NP

            def chunk_body(ci, _):
                cbase = stage_chunk(ci)
                gglo = jnp.maximum(0, (estart - cbase) // GB)
                gghi = jnp.minimum(NGB, (eend - cbase + GB - 1) // GB)

                def gb(gg, _):
                    process(cbase, gg, hbase)
                    return 0

                lax.fori_loop(gglo, gghi, gb, 0)
                return 0

            lax.fori_loop(cstart, cend, chunk_body, 0)

        def writeback(hh):
            def wb(r, _):
                for c in range(D // 16):
                    v = acc[r, pl.ds(c * 16, 16)] + biasv[pl.ds(hh * D + c * 16, 16)]
                    if apply_elu:
                        v = jnp.where(v > 0, v, jnp.exp(jnp.minimum(v, 0.0)) - 1.0)
                    acc[r, pl.ds(c * 16, 16)] = v
                return 0

            lax.fori_loop(0, ROWS, wb, 0)
            pltpu.sync_copy(acc, out_hbm.at[hh, pl.ds(base, ROWS)])

        def finalize_mean():
            scale = jnp.float32(1.0 / H)

            def wb(r, _):
                for c in range(D // 16):
                    bsum = biasv[pl.ds(c * 16, 16)]
                    for hh in range(1, H):
                        bsum = bsum + biasv[pl.ds(hh * D + c * 16, 16)]
                    acc[r, pl.ds(c * 16, 16)] = (
                        acc[r, pl.ds(c * 16, 16)] + bsum) * scale
                return 0

            lax.fori_loop(0, ROWS, wb, 0)
            pltpu.sync_copy(acc, out_hbm.at[pl.ds(base, ROWS)])

        def head_pass(hh):
            pltpu.sync_copy(el_hbm.at[pl.ds(hh * NP, NP)], elv)
            pltpu.sync_copy(er_hbm.at[pl.ds(hh * NP + base, ROWS)], erv)
            if not mean_heads:
                zero_acc()
            pass_a(hh)
            pass_b(hh)
            if not mean_heads:
                writeback(hh)
            return 0

        if mean_heads:
            zero_acc()
            lax.fori_loop(0, H, lambda hh, _: head_pass(hh), 0)
            finalize_mean()
        else:
            lax.fori_loop(0, H, lambda hh, _: head_pass(hh), 0)

    return k


def _layer(x, W, al, ar, b, src_s, dst_s, bounds, apply_elu, mean_heads):
    H = al.shape[0]
    h, el, er = _proj(x, W, al, ar)
    hflat = h.reshape(H * NP, D)
    k = _edge_sc(H, apply_elu, mean_heads)
    out = k(src_s, dst_s, bounds, el.reshape(H * NP), er.reshape(H * NP),
            hflat, b.reshape(H * D))
    if not mean_heads:
        out = jnp.transpose(out, (1, 0, 2)).reshape(NP, H * D)
    return out


def kernel(inputs, edge_index, W1, al1, ar1, b1, W2, al2, ar2, b2,
           W3, al3, ar3, b3):
    src = edge_index[0].astype(jnp.int32)
    dst = edge_index[1].astype(jnp.int32)
    order = jnp.argsort(dst)
    src_s = jnp.concatenate(
        [src[order], jnp.zeros((EP - E,), jnp.int32)])
    dst_s0 = dst[order]
    dst_s = jnp.concatenate(
        [dst_s0, jnp.full((EP - E,), N - 1, jnp.int32)])
    bounds = jnp.searchsorted(
        dst_s0, jnp.arange(NW + 1, dtype=jnp.int32) * ROWS).astype(jnp.int32)
    bounds = jnp.concatenate(
        [bounds, jnp.full((64 - NW - 1,), E, jnp.int32)])

    x = jnp.pad(inputs, ((0, NP - N), (0, 0)))
    h1 = _layer(x, W1, al1, ar1, b1, src_s, dst_s, bounds, True, False)
    h2 = _layer(h1, W2, al2, ar2, b2, src_s, dst_s, bounds, True, False)
    h3 = _layer(h2, W3, al3, ar3, b3, src_s, dst_s, bounds, False, True)
    return h3[:N]


# speedup vs baseline: 8.8677x; 1.0377x over previous
import functools

import jax
import jax.numpy as jnp
from jax import lax
from jax.experimental import pallas as pl
from jax.experimental.pallas import tpu as pltpu
from jax.experimental.pallas import tpu_sc as plsc

N = 10000
NP = 10240
E = 320000
NW = 32
ROWS = NP // NW
CH = 2048
EP = ((E + CH - 1) // CH) * CH
GB = 128
GBG = GB // 16
NGB = CH // GB
D = 128


def _proj(x, W, al, ar):
    H = al.shape[0]
    Din = x.shape[1]
    BN = 256

    def body(x_ref, w_ref, al_ref, ar_ref, h_ref, el_ref, er_ref):
        xb = x_ref[...]
        for hh in range(H):
            hb = jnp.dot(xb, w_ref[:, hh * D:(hh + 1) * D],
                         preferred_element_type=jnp.float32)
            h_ref[hh] = hb
            el_ref[hh] = jnp.sum(hb * al_ref[hh][None], axis=1)
            er_ref[hh] = jnp.sum(hb * ar_ref[hh][None], axis=1)

    return pl.pallas_call(
        body,
        grid=(NP // BN,),
        in_specs=[
            pl.BlockSpec((BN, Din), lambda i: (i, 0)),
            pl.BlockSpec((Din, H * D), lambda i: (0, 0)),
            pl.BlockSpec((H, D), lambda i: (0, 0)),
            pl.BlockSpec((H, D), lambda i: (0, 0)),
        ],
        out_specs=(
            pl.BlockSpec((H, BN, D), lambda i: (0, i, 0)),
            pl.BlockSpec((H, BN), lambda i: (0, i)),
            pl.BlockSpec((H, BN), lambda i: (0, i)),
        ),
        out_shape=(
            jax.ShapeDtypeStruct((H, NP, D), jnp.float32),
            jax.ShapeDtypeStruct((H, NP), jnp.float32),
            jax.ShapeDtypeStruct((H, NP), jnp.float32),
        ),
    )(x, W, al, ar)


def _edge_sc(H, apply_elu, mean_heads):
    oshape = (NP, D) if mean_heads else (H, NP, D)
    mesh = plsc.VectorSubcoreMesh(core_axis_name="c", subcore_axis_name="s")
    iota16 = lambda: lax.iota(jnp.int32, 16)

    @functools.partial(
        pl.kernel,
        out_type=jax.ShapeDtypeStruct(oshape, jnp.float32),
        mesh=mesh,
        compiler_params=pltpu.CompilerParams(needs_layout_passes=False),
        scratch_types=[
            pltpu.VMEM((NP,), jnp.float32),
            pltpu.VMEM((ROWS,), jnp.float32),
            pltpu.VMEM((ROWS,), jnp.float32),
            pltpu.VMEM((ROWS, D), jnp.float32),
            pltpu.VMEM((CH,), jnp.int32),
            pltpu.VMEM((CH,), jnp.int32),
            pltpu.VMEM((GB,), jnp.int32),
            pltpu.VMEM((GB,), jnp.int32),
            pltpu.VMEM((GB, D), jnp.float32),
            pltpu.VMEM((GB, D), jnp.float32),
            pltpu.VMEM((64,), jnp.int32),
            pltpu.VMEM((H * D,), jnp.float32),
            pltpu.SemaphoreType.DMA,
            pltpu.SemaphoreType.DMA,
        ],
    )
    def k(src_hbm, dst_hbm, bounds_hbm, el_hbm, er_hbm, h_hbm, b_hbm, out_hbm,
          elv, erv, denv, acc, srcv, dstv, idx0, idx1, gb0, gb1,
          boundsv, biasv, sem0, sem1):
        idxs = (idx0, idx1)
        gbufs = (gb0, gb1)
        sems = (sem0, sem1)
        wid = lax.axis_index("s") * 2 + lax.axis_index("c")
        base = wid * ROWS
        pltpu.sync_copy(bounds_hbm, boundsv)
        pltpu.sync_copy(b_hbm, biasv)
        bv = boundsv[pl.ds(wid, 16)]
        estart = bv[0]
        eend = bv[1]
        cstart = estart // CH
        cend = (eend + CH - 1) // CH

        def zero_acc():
            z = jnp.zeros((16,), jnp.float32)

            def zb(r, _):
                for c in range(D // 16):
                    acc[r, pl.ds(c * 16, 16)] = z
                return 0

            lax.fori_loop(0, ROWS, zb, 0)

        def stage_chunk(ci):
            cbase = ci * CH
            pltpu.sync_copy(src_hbm.at[pl.ds(cbase, CH)], srcv)
            pltpu.sync_copy(dst_hbm.at[pl.ds(cbase, CH)], dstv)
            return cbase

        def edge_group(cbase, off):
            s16 = srcv[pl.ds(off, 16)]
            d16 = dstv[pl.ds(off, 16)]
            gi = cbase + off + iota16()
            msk = (gi >= estart) & (gi < eend)
            dl = jnp.clip(d16 - base, 0, ROWS - 1)
            sc = jnp.clip(s16, 0, N - 1)
            ev = plsc.load_gather(elv, [sc])
            rv = plsc.load_gather(erv, [dl])
            e = ev + rv
            e = jnp.where(e > 0, e, 0.2 * e)
            return msk, dl, jnp.exp(e)

        def pass_a(_hh):
            z = jnp.zeros((16,), jnp.float32)
            for r in range(ROWS // 16):
                denv[pl.ds(r * 16, 16)] = z

            def chunk_body(ci, _):
                cbase = stage_chunk(ci)
                glo = jnp.maximum(0, (estart - cbase) // 16)
                ghi = jnp.minimum(CH // 16, (eend - cbase + 15) // 16)

                def gb(g, _):
                    msk, dl, ex = edge_group(cbase, g * 16)
                    plsc.addupdate_scatter(denv, [dl], ex, mask=msk)
                    return 0

                lax.fori_loop(glo, ghi, gb, 0)
                return 0

            lax.fori_loop(cstart, cend, chunk_body, 0)

        def issue(hbase, gg, slot):
            goff = gg * GB
            for u in range(GBG):
                s16 = srcv[pl.ds(goff + u * 16, 16)]
                idxs[slot][pl.ds(u * 16, 16)] = jnp.clip(s16, 0, N - 1) + hbase
            pltpu.async_copy(h_hbm.at[idxs[slot]], gbufs[slot], sems[slot])

        def wait(slot):
            pltpu.make_async_copy(h_hbm.at[idxs[slot]], gbufs[slot],
                                  sems[slot]).wait()

        def process(cbase, gg, slot):
            goff = gg * GB
            gbuf = gbufs[slot]
            for u in range(GBG):
                off = goff + u * 16
                msk, dl, ex = edge_group(cbase, off)
                den = plsc.load_gather(denv, [dl])
                alv = jnp.where(msk, ex / den, 0.0)
                for j in range(16):
                    a = alv[j]
                    dls = dl[j]
                    for c in range(D // 16):
                        plsc.addupdate(
                            acc.at[dls, pl.ds(c * 16, 16)],
                            a * gbuf[u * 16 + j, pl.ds(c * 16, 16)])

        def pass_b(hh):
            hbase = hh * NP

            def chunk_body(ci, _):
                cbase = stage_chunk(ci)
                gglo = jnp.maximum(0, (estart - cbase) // GB)
                gghi = jnp.minimum(NGB, (eend - cbase + GB - 1) // GB)
                for b in range(2):
                    @pl.when(gglo + b < gghi)
                    def _():
                        issue(hbase, gglo + b, b)

                def rb(r, _):
                    it = r * 2
                    for b in range(2):
                        gg = gglo + it + b

                        @pl.when(gg < gghi)
                        def _():
                            wait(b)
                            process(cbase, gg, b)

                            @pl.when(gg + 2 < gghi)
                            def _():
                                issue(hbase, gg + 2, b)
                    return 0

                nrounds = (gghi - gglo + 1) // 2
                lax.fori_loop(0, nrounds, rb, 0)
                return 0

            lax.fori_loop(cstart, cend, chunk_body, 0)

        def writeback(hh):
            def wb(r, _):
                for c in range(D // 16):
                    v = acc[r, pl.ds(c * 16, 16)] + biasv[pl.ds(hh * D + c * 16, 16)]
                    if apply_elu:
                        v = jnp.where(v > 0, v, jnp.exp(jnp.minimum(v, 0.0)) - 1.0)
                    acc[r, pl.ds(c * 16, 16)] = v
                return 0

            lax.fori_loop(0, ROWS, wb, 0)
            pltpu.sync_copy(acc, out_hbm.at[hh, pl.ds(base, ROWS)])

        def finalize_mean():
            scale = jnp.float32(1.0 / H)

            def wb(r, _):
                for c in range(D // 16):
                    bsum = biasv[pl.ds(c * 16, 16)]
                    for hh in range(1, H):
                        bsum = bsum + biasv[pl.ds(hh * D + c * 16, 16)]
                    acc[r, pl.ds(c * 16, 16)] = (
                        acc[r, pl.ds(c * 16, 16)] + bsum) * scale
                return 0

            lax.fori_loop(0, ROWS, wb, 0)
            pltpu.sync_copy(acc, out_hbm.at[pl.ds(base, ROWS)])

        def head_pass(hh):
            pltpu.sync_copy(el_hbm.at[pl.ds(hh * NP, NP)], elv)
            pltpu.sync_copy(er_hbm.at[pl.ds(hh * NP + base, ROWS)], erv)
            if not mean_heads:
                zero_acc()
            pass_a(hh)
            pass_b(hh)
            if not mean_heads:
                writeback(hh)
            return 0

        if mean_heads:
            zero_acc()
            lax.fori_loop(0, H, lambda hh, _: head_pass(hh), 0)
            finalize_mean()
        else:
            lax.fori_loop(0, H, lambda hh, _: head_pass(hh), 0)

    return k


def _layer(x, W, al, ar, b, src_s, dst_s, bounds, apply_elu, mean_heads):
    H = al.shape[0]
    h, el, er = _proj(x, W, al, ar)
    hflat = h.reshape(H * NP, D)
    k = _edge_sc(H, apply_elu, mean_heads)
    out = k(src_s, dst_s, bounds, el.reshape(H * NP), er.reshape(H * NP),
            hflat, b.reshape(H * D))
    if not mean_heads:
        out = jnp.transpose(out, (1, 0, 2)).reshape(NP, H * D)
    return out


def kernel(inputs, edge_index, W1, al1, ar1, b1, W2, al2, ar2, b2,
           W3, al3, ar3, b3):
    src = edge_index[0].astype(jnp.int32)
    dst = edge_index[1].astype(jnp.int32)
    order = jnp.argsort(dst)
    src_s = jnp.concatenate(
        [src[order], jnp.zeros((EP - E,), jnp.int32)])
    dst_s0 = dst[order]
    dst_s = jnp.concatenate(
        [dst_s0, jnp.full((EP - E,), N - 1, jnp.int32)])
    bounds = jnp.searchsorted(
        dst_s0, jnp.arange(NW + 1, dtype=jnp.int32) * ROWS).astype(jnp.int32)
    bounds = jnp.concatenate(
        [bounds, jnp.full((64 - NW - 1,), E, jnp.int32)])

    x = jnp.pad(inputs, ((0, NP - N), (0, 0)))
    h1 = _layer(x, W1, al1, ar1, b1, src_s, dst_s, bounds, True, False)
    h2 = _layer(h1, W2, al2, ar2, b2, src_s, dst_s, bounds, True, False)
    h3 = _layer(h2, W3, al3, ar3, b3, src_s, dst_s, bounds, False, True)
    return h3[:N]


# speedup vs baseline: 12.4123x; 1.3997x over previous
import functools

import jax
import jax.numpy as jnp
from jax import lax
from jax.experimental import pallas as pl
from jax.experimental.pallas import tpu as pltpu
from jax.experimental.pallas import tpu_sc as plsc

N = 10000
NP = 10240
E = 320000
NW = 32
ROWS = NP // NW
CH = 2048
EP = ((E + CH - 1) // CH) * CH
GB = 256
GBG = GB // 16
NGB = CH // GB
D = 128


def _proj(x, W, al, ar):
    H = al.shape[0]
    Din = x.shape[1]
    BN = 256

    def body(x_ref, w_ref, al_ref, ar_ref, h_ref, el_ref, er_ref):
        xb = x_ref[...]
        for hh in range(H):
            hb = jnp.dot(xb, w_ref[:, hh * D:(hh + 1) * D],
                         preferred_element_type=jnp.float32)
            h_ref[hh] = hb
            el_ref[hh] = jnp.sum(hb * al_ref[hh][None], axis=1)
            er_ref[hh] = jnp.sum(hb * ar_ref[hh][None], axis=1)

    return pl.pallas_call(
        body,
        grid=(NP // BN,),
        in_specs=[
            pl.BlockSpec((BN, Din), lambda i: (i, 0)),
            pl.BlockSpec((Din, H * D), lambda i: (0, 0)),
            pl.BlockSpec((H, D), lambda i: (0, 0)),
            pl.BlockSpec((H, D), lambda i: (0, 0)),
        ],
        out_specs=(
            pl.BlockSpec((H, BN, D), lambda i: (0, i, 0)),
            pl.BlockSpec((H, BN), lambda i: (0, i)),
            pl.BlockSpec((H, BN), lambda i: (0, i)),
        ),
        out_shape=(
            jax.ShapeDtypeStruct((H, NP, D), jnp.float32),
            jax.ShapeDtypeStruct((H, NP), jnp.float32),
            jax.ShapeDtypeStruct((H, NP), jnp.float32),
        ),
    )(x, W, al, ar)


def _edge_sc(H, apply_elu, mean_heads):
    oshape = (NP, D) if mean_heads else (H, NP, D)
    mesh = plsc.VectorSubcoreMesh(core_axis_name="c", subcore_axis_name="s")
    iota16 = lambda: lax.iota(jnp.int32, 16)

    @functools.partial(
        pl.kernel,
        out_type=jax.ShapeDtypeStruct(oshape, jnp.float32),
        mesh=mesh,
        compiler_params=pltpu.CompilerParams(needs_layout_passes=False),
        scratch_types=[
            pltpu.VMEM((NP,), jnp.float32),
            pltpu.VMEM((ROWS,), jnp.float32),
            pltpu.VMEM((ROWS,), jnp.float32),
            pltpu.VMEM((ROWS, D), jnp.float32),
            pltpu.VMEM((CH,), jnp.int32),
            pltpu.VMEM((CH,), jnp.int32),
            pltpu.VMEM((GB,), jnp.int32),
            pltpu.VMEM((GB,), jnp.int32),
            pltpu.VMEM((GB, D), jnp.float32),
            pltpu.VMEM((GB, D), jnp.float32),
            pltpu.VMEM((64,), jnp.int32),
            pltpu.VMEM((H * D,), jnp.float32),
            pltpu.SemaphoreType.DMA,
            pltpu.SemaphoreType.DMA,
        ],
    )
    def k(src_hbm, dst_hbm, bounds_hbm, el_hbm, er_hbm, h_hbm, b_hbm, out_hbm,
          elv, erv, denv, acc, srcv, dstv, idx0, idx1, gb0, gb1,
          boundsv, biasv, sem0, sem1):
        idxs = (idx0, idx1)
        gbufs = (gb0, gb1)
        sems = (sem0, sem1)
        wid = lax.axis_index("s") * 2 + lax.axis_index("c")
        base = wid * ROWS
        pltpu.sync_copy(bounds_hbm, boundsv)
        pltpu.sync_copy(b_hbm, biasv)
        bv = boundsv[pl.ds(wid, 16)]
        estart = bv[0]
        eend = bv[1]
        cstart = estart // CH
        cend = (eend + CH - 1) // CH

        def zero_acc():
            z = jnp.zeros((16,), jnp.float32)

            def zb(r, _):
                for c in range(D // 16):
                    acc[r, pl.ds(c * 16, 16)] = z
                return 0

            lax.fori_loop(0, ROWS, zb, 0)

        def stage_chunk(ci):
            cbase = ci * CH
            pltpu.sync_copy(src_hbm.at[pl.ds(cbase, CH)], srcv)
            pltpu.sync_copy(dst_hbm.at[pl.ds(cbase, CH)], dstv)
            return cbase

        def edge_group(cbase, off):
            s16 = srcv[pl.ds(off, 16)]
            d16 = dstv[pl.ds(off, 16)]
            gi = cbase + off + iota16()
            msk = (gi >= estart) & (gi < eend)
            dl = jnp.clip(d16 - base, 0, ROWS - 1)
            sc = jnp.clip(s16, 0, N - 1)
            ev = plsc.load_gather(elv, [sc])
            rv = plsc.load_gather(erv, [dl])
            e = ev + rv
            e = jnp.where(e > 0, e, 0.2 * e)
            return msk, dl, jnp.exp(e)

        def pass_a(_hh):
            z = jnp.zeros((16,), jnp.float32)
            for r in range(ROWS // 16):
                denv[pl.ds(r * 16, 16)] = z

            def chunk_body(ci, _):
                cbase = stage_chunk(ci)
                glo = jnp.maximum(0, (estart - cbase) // 16)
                ghi = jnp.minimum(CH // 16, (eend - cbase + 15) // 16)

                def gb(g, _):
                    msk, dl, ex = edge_group(cbase, g * 16)
                    plsc.addupdate_scatter(denv, [dl], ex, mask=msk)
                    return 0

                lax.fori_loop(glo, ghi, gb, 0)
                return 0

            lax.fori_loop(cstart, cend, chunk_body, 0)

        def issue(hbase, gg, slot):
            goff = gg * GB

            def ib(u, _):
                s16 = srcv[pl.ds(goff + u * 16, 16)]
                idxs[slot][pl.ds(u * 16, 16)] = jnp.clip(s16, 0, N - 1) + hbase
                return 0

            lax.fori_loop(0, GBG, ib, 0)
            pltpu.async_copy(h_hbm.at[idxs[slot]], gbufs[slot], sems[slot])

        def wait(slot):
            pltpu.make_async_copy(h_hbm.at[idxs[slot]], gbufs[slot],
                                  sems[slot]).wait()

        def process(cbase, gg, slot):
            goff = gg * GB
            gbuf = gbufs[slot]

            def ub(u, _):
                off = goff + u * 16
                msk, dl, ex = edge_group(cbase, off)
                den = plsc.load_gather(denv, [dl])
                alv = jnp.where(msk, ex / den, 0.0)
                for j in range(16):
                    a = alv[j]
                    dls = dl[j]
                    for c in range(D // 16):
                        plsc.addupdate(
                            acc.at[dls, pl.ds(c * 16, 16)],
                            a * gbuf[u * 16 + j, pl.ds(c * 16, 16)])
                return 0

            lax.fori_loop(0, GBG, ub, 0)

        def pass_b(hh):
            hbase = hh * NP

            def chunk_body(ci, _):
                cbase = stage_chunk(ci)
                gglo = jnp.maximum(0, (estart - cbase) // GB)
                gghi = jnp.minimum(NGB, (eend - cbase + GB - 1) // GB)
                for b in range(2):
                    @pl.when(gglo + b < gghi)
                    def _():
                        issue(hbase, gglo + b, b)

                def rb(r, _):
                    it = r * 2
                    for b in range(2):
                        gg = gglo + it + b

                        @pl.when(gg < gghi)
                        def _():
                            wait(b)
                            process(cbase, gg, b)

                            @pl.when(gg + 2 < gghi)
                            def _():
                                issue(hbase, gg + 2, b)
                    return 0

                nrounds = (gghi - gglo + 1) // 2
                lax.fori_loop(0, nrounds, rb, 0)
                return 0

            lax.fori_loop(cstart, cend, chunk_body, 0)

        def writeback(hh):
            def wb(r, _):
                for c in range(D // 16):
                    v = acc[r, pl.ds(c * 16, 16)] + biasv[pl.ds(hh * D + c * 16, 16)]
                    if apply_elu:
                        v = jnp.where(v > 0, v, jnp.exp(jnp.minimum(v, 0.0)) - 1.0)
                    acc[r, pl.ds(c * 16, 16)] = v
                return 0

            lax.fori_loop(0, ROWS, wb, 0)
            pltpu.sync_copy(acc, out_hbm.at[hh, pl.ds(base, ROWS)])

        def finalize_mean():
            scale = jnp.float32(1.0 / H)

            def wb(r, _):
                for c in range(D // 16):
                    bsum = biasv[pl.ds(c * 16, 16)]
                    for hh in range(1, H):
                        bsum = bsum + biasv[pl.ds(hh * D + c * 16, 16)]
                    acc[r, pl.ds(c * 16, 16)] = (
                        acc[r, pl.ds(c * 16, 16)] + bsum) * scale
                return 0

            lax.fori_loop(0, ROWS, wb, 0)
            pltpu.sync_copy(acc, out_hbm.at[pl.ds(base, ROWS)])

        def head_pass(hh):
            pltpu.sync_copy(el_hbm.at[pl.ds(hh * NP, NP)], elv)
            pltpu.sync_copy(er_hbm.at[pl.ds(hh * NP + base, ROWS)], erv)
            if not mean_heads:
                zero_acc()
            pass_a(hh)
            pass_b(hh)
            if not mean_heads:
                writeback(hh)
            return 0

        if mean_heads:
            zero_acc()
            lax.fori_loop(0, H, lambda hh, _: head_pass(hh), 0)
            finalize_mean()
        else:
            lax.fori_loop(0, H, lambda hh, _: head_pass(hh), 0)

    return k


def _layer(x, W, al, ar, b, src_s, dst_s, bounds, apply_elu, mean_heads):
    H = al.shape[0]
    h, el, er = _proj(x, W, al, ar)
    hflat = h.reshape(H * NP, D)
    k = _edge_sc(H, apply_elu, mean_heads)
    out = k(src_s, dst_s, bounds, el.reshape(H * NP), er.reshape(H * NP),
            hflat, b.reshape(H * D))
    if not mean_heads:
        out = jnp.transpose(out, (1, 0, 2)).reshape(NP, H * D)
    return out


def kernel(inputs, edge_index, W1, al1, ar1, b1, W2, al2, ar2, b2,
           W3, al3, ar3, b3):
    src = edge_index[0].astype(jnp.int32)
    dst = edge_index[1].astype(jnp.int32)
    order = jnp.argsort(dst)
    src_s = jnp.concatenate(
        [src[order], jnp.zeros((EP - E,), jnp.int32)])
    dst_s0 = dst[order]
    dst_s = jnp.concatenate(
        [dst_s0, jnp.full((EP - E,), N - 1, jnp.int32)])
    bounds = jnp.searchsorted(
        dst_s0, jnp.arange(NW + 1, dtype=jnp.int32) * ROWS).astype(jnp.int32)
    bounds = jnp.concatenate(
        [bounds, jnp.full((64 - NW - 1,), E, jnp.int32)])

    x = jnp.pad(inputs, ((0, NP - N), (0, 0)))
    h1 = _layer(x, W1, al1, ar1, b1, src_s, dst_s, bounds, True, False)
    h2 = _layer(h1, W2, al2, ar2, b2, src_s, dst_s, bounds, True, False)
    h3 = _layer(h2, W3, al3, ar3, b3, src_s, dst_s, bounds, False, True)
    return h3[:N]


# speedup vs baseline: 12.7060x; 1.0237x over previous
import functools

import jax
import jax.numpy as jnp
from jax import lax
from jax.experimental import pallas as pl
from jax.experimental.pallas import tpu as pltpu
from jax.experimental.pallas import tpu_sc as plsc

N = 10000
NP = 10240
E = 320000
NW = 32
ROWS = NP // NW
CH = 4096
EP = ((E + CH - 1) // CH) * CH
GB = 256
GBG = GB // 16
NGB = CH // GB
D = 128


def _proj(x, W, al, ar):
    H = al.shape[0]
    Din = x.shape[1]
    BN = 256

    def body(x_ref, w_ref, al_ref, ar_ref, h_ref, el_ref, er_ref):
        xb = x_ref[...]
        for hh in range(H):
            hb = jnp.dot(xb, w_ref[:, hh * D:(hh + 1) * D],
                         preferred_element_type=jnp.float32)
            h_ref[hh] = hb
            el_ref[hh] = jnp.sum(hb * al_ref[hh][None], axis=1)
            er_ref[hh] = jnp.sum(hb * ar_ref[hh][None], axis=1)

    return pl.pallas_call(
        body,
        grid=(NP // BN,),
        in_specs=[
            pl.BlockSpec((BN, Din), lambda i: (i, 0)),
            pl.BlockSpec((Din, H * D), lambda i: (0, 0)),
            pl.BlockSpec((H, D), lambda i: (0, 0)),
            pl.BlockSpec((H, D), lambda i: (0, 0)),
        ],
        out_specs=(
            pl.BlockSpec((H, BN, D), lambda i: (0, i, 0)),
            pl.BlockSpec((H, BN), lambda i: (0, i)),
            pl.BlockSpec((H, BN), lambda i: (0, i)),
        ),
        out_shape=(
            jax.ShapeDtypeStruct((H, NP, D), jnp.float32),
            jax.ShapeDtypeStruct((H, NP), jnp.float32),
            jax.ShapeDtypeStruct((H, NP), jnp.float32),
        ),
    )(x, W, al, ar)


def _edge_sc(H, apply_elu, mean_heads):
    oshape = (NP, D) if mean_heads else (H, NP, D)
    mesh = plsc.VectorSubcoreMesh(core_axis_name="c", subcore_axis_name="s")
    iota16 = lambda: lax.iota(jnp.int32, 16)

    @functools.partial(
        pl.kernel,
        out_type=jax.ShapeDtypeStruct(oshape, jnp.float32),
        mesh=mesh,
        compiler_params=pltpu.CompilerParams(needs_layout_passes=False),
        scratch_types=[
            pltpu.VMEM((NP,), jnp.float32),
            pltpu.VMEM((ROWS,), jnp.float32),
            pltpu.VMEM((ROWS,), jnp.float32),
            pltpu.VMEM((ROWS, D), jnp.float32),
            pltpu.VMEM((CH,), jnp.int32),
            pltpu.VMEM((CH,), jnp.int32),
            pltpu.VMEM((GB,), jnp.int32),
            pltpu.VMEM((GB,), jnp.int32),
            pltpu.VMEM((GB, D), jnp.float32),
            pltpu.VMEM((GB, D), jnp.float32),
            pltpu.VMEM((64,), jnp.int32),
            pltpu.VMEM((H * D,), jnp.float32),
            pltpu.SemaphoreType.DMA,
            pltpu.SemaphoreType.DMA,
        ],
    )
    def k(src_hbm, dst_hbm, bounds_hbm, el_hbm, er_hbm, h_hbm, b_hbm, out_hbm,
          elv, erv, denv, acc, srcv, dstv, idx0, idx1, gb0, gb1,
          boundsv, biasv, sem0, sem1):
        idxs = (idx0, idx1)
        gbufs = (gb0, gb1)
        sems = (sem0, sem1)
        wid = lax.axis_index("s") * 2 + lax.axis_index("c")
        base = wid * ROWS
        pltpu.sync_copy(bounds_hbm, boundsv)
        pltpu.sync_copy(b_hbm, biasv)
        bv = boundsv[pl.ds(wid, 16)]
        estart = bv[0]
        eend = bv[1]
        cstart = estart // CH
        cend = (eend + CH - 1) // CH

        def zero_acc():
            z = jnp.zeros((16,), jnp.float32)

            def zb(r, _):
                for c in range(D // 16):
                    acc[r, pl.ds(c * 16, 16)] = z
                return 0

            lax.fori_loop(0, ROWS, zb, 0)

        def stage_chunk(ci):
            cbase = ci * CH
            pltpu.sync_copy(src_hbm.at[pl.ds(cbase, CH)], srcv)
            pltpu.sync_copy(dst_hbm.at[pl.ds(cbase, CH)], dstv)
            return cbase

        def edge_group(cbase, off):
            s16 = srcv[pl.ds(off, 16)]
            d16 = dstv[pl.ds(off, 16)]
            gi = cbase + off + iota16()
            msk = (gi >= estart) & (gi < eend)
            dl = jnp.clip(d16 - base, 0, ROWS - 1)
            sc = jnp.clip(s16, 0, N - 1)
            ev = plsc.load_gather(elv, [sc])
            rv = plsc.load_gather(erv, [dl])
            e = ev + rv
            e = jnp.where(e > 0, e, 0.2 * e)
            return msk, dl, jnp.exp(e)

        def pass_a(_hh):
            z = jnp.zeros((16,), jnp.float32)
            for r in range(ROWS // 16):
                denv[pl.ds(r * 16, 16)] = z

            def chunk_body(ci, _):
                cbase = stage_chunk(ci)
                glo = jnp.maximum(0, (estart - cbase) // 16)
                ghi = jnp.minimum(CH // 16, (eend - cbase + 15) // 16)

                def gb(g, _):
                    msk, dl, ex = edge_group(cbase, g * 16)
                    plsc.addupdate_scatter(denv, [dl], ex, mask=msk)
                    return 0

                lax.fori_loop(glo, ghi, gb, 0)
                return 0

            lax.fori_loop(cstart, cend, chunk_body, 0)

        def issue(hbase, gg, slot):
            goff = gg * GB

            def ib(u, _):
                s16 = srcv[pl.ds(goff + u * 16, 16)]
                idxs[slot][pl.ds(u * 16, 16)] = jnp.clip(s16, 0, N - 1) + hbase
                return 0

            lax.fori_loop(0, GBG, ib, 0)
            pltpu.async_copy(h_hbm.at[idxs[slot]], gbufs[slot], sems[slot])

        def wait(slot):
            pltpu.make_async_copy(h_hbm.at[idxs[slot]], gbufs[slot],
                                  sems[slot]).wait()

        def process(cbase, gg, slot):
            goff = gg * GB
            gbuf = gbufs[slot]

            def ub(u, _):
                off = goff + u * 16
                msk, dl, ex = edge_group(cbase, off)
                den = plsc.load_gather(denv, [dl])
                alv = jnp.where(msk, ex / den, 0.0)
                for j in range(16):
                    a = alv[j]
                    dls = dl[j]
                    for c in range(D // 16):
                        plsc.addupdate(
                            acc.at[dls, pl.ds(c * 16, 16)],
                            a * gbuf[u * 16 + j, pl.ds(c * 16, 16)])
                return 0

            lax.fori_loop(0, GBG, ub, 0)

        def pass_b(hh):
            hbase = hh * NP

            def chunk_body(ci, _):
                cbase = stage_chunk(ci)
                gglo = jnp.maximum(0, (estart - cbase) // GB)
                gghi = jnp.minimum(NGB, (eend - cbase + GB - 1) // GB)
                for b in range(2):
                    @pl.when(gglo + b < gghi)
                    def _():
                        issue(hbase, gglo + b, b)

                def rb(r, _):
                    it = r * 2
                    for b in range(2):
                        gg = gglo + it + b

                        @pl.when(gg < gghi)
                        def _():
                            wait(b)
                            process(cbase, gg, b)

                            @pl.when(gg + 2 < gghi)
                            def _():
                                issue(hbase, gg + 2, b)
                    return 0

                nrounds = (gghi - gglo + 1) // 2
                lax.fori_loop(0, nrounds, rb, 0)
                return 0

            lax.fori_loop(cstart, cend, chunk_body, 0)

        def writeback(hh):
            def wb(r, _):
                for c in range(D // 16):
                    v = acc[r, pl.ds(c * 16, 16)] + biasv[pl.ds(hh * D + c * 16, 16)]
                    if apply_elu:
                        v = jnp.where(v > 0, v, jnp.exp(jnp.minimum(v, 0.0)) - 1.0)
                    acc[r, pl.ds(c * 16, 16)] = v
                return 0

            lax.fori_loop(0, ROWS, wb, 0)
            pltpu.sync_copy(acc, out_hbm.at[hh, pl.ds(base, ROWS)])

        def finalize_mean():
            scale = jnp.float32(1.0 / H)

            def wb(r, _):
                for c in range(D // 16):
                    bsum = biasv[pl.ds(c * 16, 16)]
                    for hh in range(1, H):
                        bsum = bsum + biasv[pl.ds(hh * D + c * 16, 16)]
                    acc[r, pl.ds(c * 16, 16)] = (
                        acc[r, pl.ds(c * 16, 16)] + bsum) * scale
                return 0

            lax.fori_loop(0, ROWS, wb, 0)
            pltpu.sync_copy(acc, out_hbm.at[pl.ds(base, ROWS)])

        def head_pass(hh):
            pltpu.sync_copy(el_hbm.at[pl.ds(hh * NP, NP)], elv)
            pltpu.sync_copy(er_hbm.at[pl.ds(hh * NP + base, ROWS)], erv)
            if not mean_heads:
                zero_acc()
            pass_a(hh)
            pass_b(hh)
            if not mean_heads:
                writeback(hh)
            return 0

        if mean_heads:
            zero_acc()
            lax.fori_loop(0, H, lambda hh, _: head_pass(hh), 0)
            finalize_mean()
        else:
            lax.fori_loop(0, H, lambda hh, _: head_pass(hh), 0)

    return k


def _layer(x, W, al, ar, b, src_s, dst_s, bounds, apply_elu, mean_heads):
    H = al.shape[0]
    h, el, er = _proj(x, W, al, ar)
    hflat = h.reshape(H * NP, D)
    k = _edge_sc(H, apply_elu, mean_heads)
    out = k(src_s, dst_s, bounds, el.reshape(H * NP), er.reshape(H * NP),
            hflat, b.reshape(H * D))
    if not mean_heads:
        out = jnp.transpose(out, (1, 0, 2)).reshape(NP, H * D)
    return out


def kernel(inputs, edge_index, W1, al1, ar1, b1, W2, al2, ar2, b2,
           W3, al3, ar3, b3):
    src = edge_index[0].astype(jnp.int32)
    dst = edge_index[1].astype(jnp.int32)
    order = jnp.argsort(dst)
    src_s = jnp.concatenate(
        [src[order], jnp.zeros((EP - E,), jnp.int32)])
    dst_s0 = dst[order]
    dst_s = jnp.concatenate(
        [dst_s0, jnp.full((EP - E,), N - 1, jnp.int32)])
    bounds = jnp.searchsorted(
        dst_s0, jnp.arange(NW + 1, dtype=jnp.int32) * ROWS).astype(jnp.int32)
    bounds = jnp.concatenate(
        [bounds, jnp.full((64 - NW - 1,), E, jnp.int32)])

    x = jnp.pad(inputs, ((0, NP - N), (0, 0)))
    h1 = _layer(x, W1, al1, ar1, b1, src_s, dst_s, bounds, True, False)
    h2 = _layer(h1, W2, al2, ar2, b2, src_s, dst_s, bounds, True, False)
    h3 = _layer(h2, W3, al3, ar3, b3, src_s, dst_s, bounds, False, True)
    return h3[:N]
